# baseline (speedup 1.0000x reference)
import functools

import jax
import jax.numpy as jnp
from jax import lax
from jax.experimental import pallas as pl
from jax.experimental.pallas import tpu as pltpu
from jax.experimental.pallas import tpu_sc as plsc

N = 10000
E = 320000
HID = 128
D_IN = 128
D_EDGE = 4
NG = 64
XW = 16
CH = 128
NCH = E // CH
NC = 2
NS = 16
NW = NC * NS
TRIPS = -(-NCH // NW)
RPS = N // NS

BE = 8000
BN = 2000
BP = 2000
GP = N // BP


def _sigmoid(v):
    return 0.5 * jnp.tanh(0.5 * v) + 0.5


def _silu(v):
    return v * _sigmoid(v)


def _sc_gather(pr, pc, x16, xneg, row, col):
    eh = row.shape[0]
    nch = eh // CH
    trips = -(-nch // NW)

    @functools.partial(
        pl.kernel,
        out_type=(
            jax.ShapeDtypeStruct((eh, HID), jnp.bfloat16),
            jax.ShapeDtypeStruct((eh, XW), jnp.float32),
        ),
        mesh=plsc.VectorSubcoreMesh(core_axis_name="c", subcore_axis_name="s"),
        scratch_types=(
            pltpu.VMEM((3, CH), jnp.int32),
            pltpu.VMEM((3, CH), jnp.int32),
            pltpu.VMEM((3, CH, HID), jnp.bfloat16),
            pltpu.VMEM((3, CH, XW), jnp.float32),
            pltpu.SemaphoreType.DMA,
            pltpu.SemaphoreType.DMA,
            pltpu.SemaphoreType.DMA,
            pltpu.SemaphoreType.DMA,
            pltpu.SemaphoreType.DMA,
            pltpu.SemaphoreType.DMA,
            pltpu.SemaphoreType.DMA,
            pltpu.SemaphoreType.DMA,
            pltpu.SemaphoreType.DMA,
        ),
        compiler_params=pltpu.CompilerParams(use_tc_tiling_on_sc=False),
    )
    def k(pr_ref, pc_ref, x_ref, xn_ref, row_ref, col_ref, z1_ref, dx_ref,
          ir3, ic3, bz, bx,
          ga0, ga1, ga2, gb0, gb1, gb2, st0, st1, st2):
        w = lax.axis_index("s") * NC + lax.axis_index("c")
        ga = (ga0, ga1, ga2)
        gb = (gb0, gb1, gb2)
        st = (st0, st1, st2)

        def valid(t):
            return (t >= 0) & (w + t * NW < nch)

        def a_start(t, b):
            base = (w + t * NW) * CH
            pltpu.sync_copy(row_ref.at[pl.ds(base, CH)], ir3.at[b])
            pltpu.sync_copy(col_ref.at[pl.ds(base, CH)], ic3.at[b])
            pltpu.async_copy(pr_ref.at[ir3.at[b]], bz.at[b], ga[b])
            pltpu.async_copy(x_ref.at[ir3.at[b]], bx.at[b], ga[b])

        def a_drain(b):
            pltpu.make_async_copy(pr_ref.at[pl.ds(0, CH)], bz.at[b], ga[b]).wait()
            pltpu.make_async_copy(x_ref.at[pl.ds(0, CH)], bx.at[b], ga[b]).wait()

        def b_start(b):
            pltpu.async_copy(pc_ref.at[ic3.at[b]], bz.at[b], gb[b], add=True)
            pltpu.async_copy(xn_ref.at[ic3.at[b]], bx.at[b], gb[b], add=True)

        def b_drain(b):
            pltpu.make_async_copy(pc_ref.at[pl.ds(0, CH)], bz.at[b], gb[b]).wait()
            pltpu.make_async_copy(xn_ref.at[pl.ds(0, CH)], bx.at[b], gb[b]).wait()

        def store_start(t, b):
            base = (w + t * NW) * CH
            pltpu.async_copy(bz.at[b], z1_ref.at[pl.ds(base, CH)], st[b])
            pltpu.async_copy(bx.at[b], dx_ref.at[pl.ds(base, CH)], st[b])

        def store_drain(b):
            pltpu.make_async_copy(bz.at[b], z1_ref.at[pl.ds(0, CH)], st[b]).wait()
            pltpu.make_async_copy(bx.at[b], dx_ref.at[pl.ds(0, CH)], st[b]).wait()

        a_start(0, 0)

        def outer(g, carry):
            for b0 in (0, 1, 2):
                t = 3 * g + b0
                bt = b0
                bp = (b0 + 2) % 3
                bq = (b0 + 1) % 3

                @pl.when(valid(t - 2))
                def _():
                    store_drain(bq)

                @pl.when(valid(t + 1))
                def _():
                    a_start(t + 1, bq)

                @pl.when(valid(t))
                def _():
                    a_drain(bt)
                    b_start(bt)

                @pl.when(valid(t - 1))
                def _():
                    b_drain(bp)
                    store_start(t - 1, bp)

            return carry

        lax.fori_loop(0, (trips + 3) // 3 + 1, outer, 0)

    return k(pr, pc, x16, xneg, row, col)


def _sc_scatter(ef, tr, row, z128, z16):
    eh = row.shape[0]
    nch = eh // CH
    trips = -(-nch // NW)

    @functools.partial(
        pl.kernel,
        out_type=(
            jax.ShapeDtypeStruct((NC, N, HID), jnp.float32),
            jax.ShapeDtypeStruct((NC, N, XW), jnp.float32),
        ),
        mesh=plsc.VectorSubcoreMesh(core_axis_name="c", subcore_axis_name="s"),
        scratch_types=(
            pltpu.VMEM_SHARED((N, HID), jnp.float32),
            pltpu.VMEM_SHARED((N, XW), jnp.float32),
            pltpu.VMEM((2, CH), jnp.int32),
            pltpu.VMEM((2, CH, HID), jnp.float32),
            pltpu.VMEM((2, CH, XW), jnp.float32),
            pltpu.SemaphoreType.DMA,
            pltpu.SemaphoreType.DMA,
            pltpu.SemaphoreType.DMA,
            pltpu.SemaphoreType.DMA,
        ),
        compiler_params=pltpu.CompilerParams(use_tc_tiling_on_sc=False),
    )
    def k(ef_ref, tr_ref, row_ref, z128_ref, z16_ref, agg_ref, tagg_ref,
          acc, tacc, ir2, bef, btr, ls0, ls1, as0, as1):
        c = lax.axis_index("c")
        s = lax.axis_index("s")
        w = s * NC + c
        r0 = s * RPS
        pltpu.sync_copy(z128_ref, acc.at[pl.ds(r0, RPS)])
        pltpu.sync_copy(z16_ref, tacc.at[pl.ds(r0, RPS)])
        plsc.subcore_barrier()
        ls = (ls0, ls1)
        am = (as0, as1)

        def valid(t):
            return (t >= 0) & (w + t * NW < nch)

        def load_start(t, b):
            base = (w + t * NW) * CH
            pltpu.async_copy(row_ref.at[pl.ds(base, CH)], ir2.at[b], ls[b])
            pltpu.async_copy(ef_ref.at[pl.ds(base, CH)], bef.at[b], ls[b])
            pltpu.async_copy(tr_ref.at[pl.ds(base, CH)], btr.at[b], ls[b])

        def load_drain(b):
            pltpu.make_async_copy(row_ref.at[pl.ds(0, CH)], ir2.at[b], ls[b]).wait()
            pltpu.make_async_copy(ef_ref.at[pl.ds(0, CH)], bef.at[b], ls[b]).wait()
            pltpu.make_async_copy(tr_ref.at[pl.ds(0, CH)], btr.at[b], ls[b]).wait()

        def add_start(b):
            pltpu.async_copy(bef.at[b], acc.at[ir2.at[b]], am[b], add=True)
            pltpu.async_copy(btr.at[b], tacc.at[ir2.at[b]], am[b], add=True)

        def add_drain(b):
            pltpu.make_async_copy(bef.at[b], acc.at[pl.ds(0, CH)], am[b]).wait()
            pltpu.make_async_copy(btr.at[b], tacc.at[pl.ds(0, CH)], am[b]).wait()

        load_start(0, 0)

        def outer(g, carry):
            for b in (0, 1):
                t = 2 * g + b
                o = 1 - b

                @pl.when(valid(t - 1))
                def _():
                    add_drain(o)

                @pl.when(valid(t + 1))
                def _():
                    load_start(t + 1, o)

                @pl.when(valid(t))
                def _():
                    load_drain(b)
                    add_start(b)

            return carry

        lax.fori_loop(0, (trips + 1) // 2, outer, 0)
        plsc.subcore_barrier()
        pltpu.sync_copy(acc.at[pl.ds(r0, RPS)], agg_ref.at[c, pl.ds(r0, RPS)])
        pltpu.sync_copy(tacc.at[pl.ds(r0, RPS)], tagg_ref.at[c, pl.ds(r0, RPS)])

    return k(ef, tr, row, z128, z16)


def _full(shape):
    return pl.BlockSpec(shape, lambda i: tuple(0 for _ in shape))


def _tc_embed(h, x16, emb_w, emb_b, w1a, w1b, b1):
    def body(h_ref, x_ref, ew_r, eb_r, w1a_r, w1b_r, b1_r,
             hh_ref, pr_ref, pc_ref, xn_ref):
        hh = jnp.dot(h_ref[...], ew_r[...], preferred_element_type=jnp.float32) + eb_r[...]
        hh_ref[...] = hh
        pr_ref[...] = (
            jnp.dot(hh, w1a_r[...], preferred_element_type=jnp.float32) + b1_r[...]
        ).astype(jnp.bfloat16)
        pc_ref[...] = jnp.dot(
            hh, w1b_r[...], preferred_element_type=jnp.float32
        ).astype(jnp.bfloat16)
        xn_ref[...] = -x_ref[...]

    return pl.pallas_call(
        body,
        grid=(N // BN,),
        in_specs=[
            pl.BlockSpec((BN, D_IN), lambda i: (i, 0)),
            pl.BlockSpec((BN, XW), lambda i: (i, 0)),
            _full((D_IN, HID)),
            _full((1, HID)),
            _full((HID, HID)),
            _full((HID, HID)),
            _full((1, HID)),
        ],
        out_specs=[
            pl.BlockSpec((BN, HID), lambda i: (i, 0)),
            pl.BlockSpec((BN, HID), lambda i: (i, 0)),
            pl.BlockSpec((BN, HID), lambda i: (i, 0)),
            pl.BlockSpec((BN, XW), lambda i: (i, 0)),
        ],
        out_shape=[
            jax.ShapeDtypeStruct((N, HID), jnp.float32),
            jax.ShapeDtypeStruct((N, HID), jnp.bfloat16),
            jax.ShapeDtypeStruct((N, HID), jnp.bfloat16),
            jax.ShapeDtypeStruct((N, XW), jnp.float32),
        ],
    )(h, x16, emb_w, emb_b.reshape(1, HID), w1a, w1b, b1.reshape(1, HID))


def _tc_edge(z1, dx, ea, wts):
    (w1r, w1e, w2, b2, aw, ab, cw1, cb1, cw2) = wts
    eh = z1.shape[0]

    def body(z1_ref, dx_ref, ea_ref,
             w1r_r, w1e_r, w2_r, b2_r, aw_r, ab_r, cw1_r, cb1_r, cw2_r,
             ef_ref, tr_ref):
        diff = dx_ref[...]
        radial = jnp.dot(diff * diff, jnp.ones((XW, 1), jnp.float32),
                         preferred_element_type=jnp.float32)
        z = (
            z1_ref[...].astype(jnp.float32)
            + jnp.dot(ea_ref[...], w1e_r[...], preferred_element_type=jnp.float32)
            + radial * w1r_r[...]
        )
        z = _silu(z)
        z = _silu(jnp.dot(z, w2_r[...], preferred_element_type=jnp.float32) + b2_r[...])
        att = _sigmoid(jnp.dot(z, aw_r[...], preferred_element_type=jnp.float32)
                       + ab_r[...])
        efv = z * att
        t = _silu(jnp.dot(efv, cw1_r[...], preferred_element_type=jnp.float32) + cb1_r[...])
        ts = jnp.tanh(jnp.dot(t, cw2_r[...], preferred_element_type=jnp.float32))
        lane = lax.broadcasted_iota(jnp.int32, (BE, XW), 1)
        ef_ref[...] = efv
        tr_ref[...] = jnp.where(lane == 3, 1.0, diff * ts)

    return pl.pallas_call(
        body,
        grid=(eh // BE,),
        in_specs=[
            pl.BlockSpec((BE, HID), lambda i: (i, 0)),
            pl.BlockSpec((BE, XW), lambda i: (i, 0)),
            pl.BlockSpec((BE, D_EDGE), lambda i: (i, 0)),
            _full((1, HID)),
            _full((D_EDGE, HID)),
            _full((HID, HID)),
            _full((1, HID)),
            _full((HID, 1)),
            _full((1, 1)),
            _full((HID, HID)),
            _full((1, HID)),
            _full((HID, 1)),
        ],
        out_specs=[
            pl.BlockSpec((BE, HID), lambda i: (i, 0)),
            pl.BlockSpec((BE, XW), lambda i: (i, 0)),
        ],
        out_shape=[
            jax.ShapeDtypeStruct((eh, HID), jnp.float32),
            jax.ShapeDtypeStruct((eh, XW), jnp.float32),
        ],
    )(z1, dx, ea, w1r, w1e, w2, b2, aw, ab, cw1, cb1, cw2)


def _tc_node(hh, x16, aggs, taggs, wts, nxt):
    (nw1a, nw1b, nb1, nw2, nb2) = wts
    last = len(nxt) == 2
    nh_parts = len(aggs)
    npart = 2 * nh_parts

    def _hh2(hh_ref, a_refs, nw1a_r, nw1b_r, nb1_r, nw2_r, nb2_r):
        aggv = a_refs[0][0]
        for a in a_refs[1:]:
            aggv = aggv + a[0]
        nh = _silu(
            jnp.dot(hh_ref[...], nw1a_r[...], preferred_element_type=jnp.float32)
            + jnp.dot(aggv, nw1b_r[...], preferred_element_type=jnp.float32)
            + nb1_r[...]
        )
        return (
            hh_ref[...]
            + jnp.dot(nh, nw2_r[...], preferred_element_type=jnp.float32)
            + nb2_r[...]
        )

    def body_last(*refs):
        hh_ref = refs[0]
        a_refs = refs[1:1 + npart]
        (x_ref, nw1a_r, nw1b_r, nb1_r, nw2_r, nb2_r, wo_r, bo_r, z_ref) = \
            refs[1 + 2 * npart:]
        hh2 = _hh2(hh_ref, a_refs, nw1a_r, nw1b_r, nb1_r, nw2_r, nb2_r)
        z_ref[...] = (
            jnp.dot(hh2, wo_r[...], preferred_element_type=jnp.float32) + bo_r[...]
        )

    def body_mid(*refs):
        hh_ref = refs[0]
        a_refs = refs[1:1 + npart]
        t_refs = refs[1 + npart:1 + 2 * npart]
        (x_ref, nw1a_r, nw1b_r, nb1_r, nw2_r, nb2_r, pa_r, pcw_r, pb_r,
         hho_ref, xo_ref, xno_ref, pr_ref, pc_ref) = refs[1 + 2 * npart:]
        hh2 = _hh2(hh_ref, a_refs, nw1a_r, nw1b_r, nb1_r, nw2_r, nb2_r)
        ts = t_refs[0][0]
        for t in t_refs[1:]:
            ts = ts + t[0]
        cnt = ts[:, 3:4]
        inv = 1.0 / jnp.maximum(cnt, 1.0)
        lane = lax.broadcasted_iota(jnp.int32, (BN, XW), 1)
        xo = x_ref[...] + jnp.where(lane < 3, ts * inv, 0.0)
        hho_ref[...] = hh2
        xo_ref[...] = xo
        xno_ref[...] = -xo
        pr_ref[...] = (
            jnp.dot(hh2, pa_r[...], preferred_element_type=jnp.float32) + pb_r[...]
        ).astype(jnp.bfloat16)
        pc_ref[...] = jnp.dot(
            hh2, pcw_r[...], preferred_element_type=jnp.float32
        ).astype(jnp.bfloat16)

    wrapped_body = body_last if last else body_mid

    def _part_spec(core, width):
        return pl.BlockSpec((1, BN, width), lambda i, c=core: (c, i, 0))

    agg_specs = []
    agg_args = []
    for a in aggs:
        for core in range(NC):
            agg_specs.append(_part_spec(core, HID))
            agg_args.append(a)
    tagg_specs = []
    tagg_args = []
    for t in taggs:
        for core in range(NC):
            tagg_specs.append(_part_spec(core, XW))
            tagg_args.append(t)

    base_in = (
        [pl.BlockSpec((BN, HID), lambda i: (i, 0))]
        + agg_specs
        + tagg_specs
        + [
            pl.BlockSpec((BN, XW), lambda i: (i, 0)),
            _full((HID, HID)),
            _full((HID, HID)),
            _full((1, HID)),
            _full((HID, HID)),
            _full((1, HID)),
        ]
    )
    base_args = [hh] + agg_args + tagg_args + [
        x16, nw1a, nw1b, nb1.reshape(1, HID), nw2, nb2.reshape(1, HID)]
    if last:
        w_out, b_out = nxt
        in_specs = base_in + [_full((HID, HID)), _full((1, HID))]
        out_specs = [pl.BlockSpec((BN, HID), lambda i: (i, 0))]
        out_shape = [jax.ShapeDtypeStruct((N, HID), jnp.float32)]
        args = tuple(base_args + [w_out, b_out.reshape(1, HID)])
    else:
        w1a_n, w1b_n, b1_n = nxt
        in_specs = base_in + [_full((HID, HID)), _full((HID, HID)), _full((1, HID))]
        out_specs = [
            pl.BlockSpec((BN, HID), lambda i: (i, 0)),
            pl.BlockSpec((BN, XW), lambda i: (i, 0)),
            pl.BlockSpec((BN, XW), lambda i: (i, 0)),
            pl.BlockSpec((BN, HID), lambda i: (i, 0)),
            pl.BlockSpec((BN, HID), lambda i: (i, 0)),
        ]
        out_shape = [
            jax.ShapeDtypeStruct((N, HID), jnp.float32),
            jax.ShapeDtypeStruct((N, XW), jnp.float32),
            jax.ShapeDtypeStruct((N, XW), jnp.float32),
            jax.ShapeDtypeStruct((N, HID), jnp.bfloat16),
            jax.ShapeDtypeStruct((N, HID), jnp.bfloat16),
        ]
        args = tuple(base_args + [w1a_n, w1b_n, b1_n.reshape(1, HID)])

    return pl.pallas_call(
        wrapped_body,
        grid=(N // BN,),
        in_specs=in_specs,
        out_specs=out_specs,
        out_shape=out_shape,
    )(*args)


def _tc_pool(z, batch3):
    def body(z_ref, bt_ref, o_ref, acc, cacc):
        i = pl.program_id(0)

        @pl.when(i == 0)
        def _():
            acc[...] = jnp.zeros_like(acc)
            cacc[...] = jnp.zeros_like(cacc)

        gid = lax.broadcasted_iota(jnp.int32, (NG, BP), 0)
        oh = (gid == bt_ref[0]).astype(jnp.float32)
        acc[...] += jnp.dot(oh, z_ref[...], preferred_element_type=jnp.float32)
        cacc[...] += jnp.sum(oh, axis=1, keepdims=True)

        @pl.when(i == GP - 1)
        def _():
            o_ref[...] = acc[...] / jnp.maximum(cacc[...], 1.0)

    return pl.pallas_call(
        body,
        grid=(GP,),
        in_specs=[
            pl.BlockSpec((BP, HID), lambda i: (i, 0)),
            pl.BlockSpec((1, 1, BP), lambda i: (i, 0, 0)),
        ],
        out_specs=pl.BlockSpec((NG, HID), lambda i: (0, 0)),
        out_shape=jax.ShapeDtypeStruct((NG, HID), jnp.float32),
        scratch_shapes=[
            pltpu.VMEM((NG, HID), jnp.float32),
            pltpu.VMEM((NG, 1), jnp.float32),
        ],
    )(z, batch3)


def _edge_split(l):
    ew1 = l["edge_w1"]
    return ew1[:HID], ew1[HID:2 * HID], ew1[2 * HID:2 * HID + 1], ew1[2 * HID + 1:]


def kernel(h, x, edge_attr, params, edges, batch):
    row = edges[0]
    col = edges[1]
    eh = E // 2
    halves = (
        (row[:eh], col[:eh], edge_attr[:eh]),
        (row[eh:], col[eh:], edge_attr[eh:]),
    )
    x16 = jnp.zeros((N, XW), jnp.float32).at[:, :3].set(x)
    z128 = jnp.zeros((RPS, HID), jnp.float32)
    z16 = jnp.zeros((RPS, XW), jnp.float32)
    batch3 = batch.reshape(GP, 1, BP)
    layers = params["layers"]

    w1a0, w1b0, _, _ = _edge_split(layers[0])
    hh, pr, pc, xneg = _tc_embed(
        h, x16, params["emb_in_w"], params["emb_in_b"], w1a0, w1b0,
        layers[0]["edge_b1"])

    for li, l in enumerate(layers):
        _, _, w1r, w1e = _edge_split(l)
        edge_wts = (
            w1r,
            w1e,
            l["edge_w2"],
            l["edge_b2"].reshape(1, HID),
            l["att_w"],
            l["att_b"].reshape(1, 1),
            l["coord_w1"],
            l["coord_b1"].reshape(1, HID),
            l["coord_w2"],
        )
        node_wts = (
            l["node_w1"][:HID],
            l["node_w1"][HID:],
            l["node_b1"],
            l["node_w2"],
            l["node_b2"],
        )
        if li + 1 < len(layers):
            nl = layers[li + 1]
            w1a_n, w1b_n, _, _ = _edge_split(nl)
            nxt = (w1a_n, w1b_n, nl["edge_b1"])
        else:
            nxt = (params["emb_out_w"], params["emb_out_b"])

        aggs = []
        taggs = []
        efts = []
        for hrow, hcol, hea in halves:
            z1, dx = _sc_gather(pr, pc, x16, xneg, hrow, hcol)
            efts.append(_tc_edge(z1, dx, hea, edge_wts))
        for (hrow, _, _), (ef, tr) in zip(halves, efts):
            agg, tagg = _sc_scatter(ef, tr, hrow, z128, z16)
            aggs.append(agg)
            taggs.append(tagg)
        outs = _tc_node(hh, x16, tuple(aggs), tuple(taggs), node_wts, nxt)
        if li + 1 < len(layers):
            hh, x16, xneg, pr, pc = outs
        else:
            (zfin,) = outs

    return _tc_pool(zfin, batch3)


# speedup vs baseline: 1.2946x; 1.2946x over previous
import functools

import jax
import jax.numpy as jnp
from jax import lax
from jax.experimental import pallas as pl
from jax.experimental.pallas import tpu as pltpu
from jax.experimental.pallas import tpu_sc as plsc

N = 10000
E = 320000
HID = 128
D_IN = 128
D_EDGE = 4
NG = 64
XW = 16
CH = 128
NCH = E // CH
NC = 2
NS = 16
NW = NC * NS
TRIPS = -(-NCH // NW)
RPS = N // NS

BE = 8000
BN = 2000
BP = 2000
GP = N // BP


def _sigmoid(v):
    return 0.5 * jnp.tanh(0.5 * v) + 0.5


def _silu(v):
    return v * _sigmoid(v)


def _sc_gather(pr, pc, x16, xneg, row, col):
    eh = row.shape[0]
    nch = eh // CH
    trips = -(-nch // NW)

    @functools.partial(
        pl.kernel,
        out_type=(
            jax.ShapeDtypeStruct((eh, HID), jnp.float32),
            jax.ShapeDtypeStruct((eh, XW), jnp.float32),
        ),
        mesh=plsc.VectorSubcoreMesh(core_axis_name="c", subcore_axis_name="s"),
        scratch_types=(
            pltpu.VMEM((3, CH), jnp.int32),
            pltpu.VMEM((3, CH), jnp.int32),
            pltpu.VMEM((3, CH, HID), jnp.float32),
            pltpu.VMEM((3, CH, XW), jnp.float32),
            pltpu.SemaphoreType.DMA,
            pltpu.SemaphoreType.DMA,
            pltpu.SemaphoreType.DMA,
            pltpu.SemaphoreType.DMA,
            pltpu.SemaphoreType.DMA,
            pltpu.SemaphoreType.DMA,
            pltpu.SemaphoreType.DMA,
            pltpu.SemaphoreType.DMA,
            pltpu.SemaphoreType.DMA,
        ),
        compiler_params=pltpu.CompilerParams(use_tc_tiling_on_sc=False),
    )
    def k(pr_ref, pc_ref, x_ref, xn_ref, row_ref, col_ref, z1_ref, dx_ref,
          ir3, ic3, bz, bx,
          ga0, ga1, ga2, gb0, gb1, gb2, st0, st1, st2):
        w = lax.axis_index("s") * NC + lax.axis_index("c")
        ga = (ga0, ga1, ga2)
        gb = (gb0, gb1, gb2)
        st = (st0, st1, st2)

        def valid(t):
            return (t >= 0) & (w + t * NW < nch)

        def a_start(t, b):
            base = (w + t * NW) * CH
            pltpu.sync_copy(row_ref.at[pl.ds(base, CH)], ir3.at[b])
            pltpu.sync_copy(col_ref.at[pl.ds(base, CH)], ic3.at[b])
            pltpu.async_copy(pr_ref.at[ir3.at[b]], bz.at[b], ga[b])
            pltpu.async_copy(x_ref.at[ir3.at[b]], bx.at[b], ga[b])

        def a_drain(b):
            pltpu.make_async_copy(pr_ref.at[pl.ds(0, CH)], bz.at[b], ga[b]).wait()
            pltpu.make_async_copy(x_ref.at[pl.ds(0, CH)], bx.at[b], ga[b]).wait()

        def b_start(b):
            pltpu.async_copy(pc_ref.at[ic3.at[b]], bz.at[b], gb[b], add=True)
            pltpu.async_copy(xn_ref.at[ic3.at[b]], bx.at[b], gb[b], add=True)

        def b_drain(b):
            pltpu.make_async_copy(pc_ref.at[pl.ds(0, CH)], bz.at[b], gb[b]).wait()
            pltpu.make_async_copy(xn_ref.at[pl.ds(0, CH)], bx.at[b], gb[b]).wait()

        def store_start(t, b):
            base = (w + t * NW) * CH
            pltpu.async_copy(bz.at[b], z1_ref.at[pl.ds(base, CH)], st[b])
            pltpu.async_copy(bx.at[b], dx_ref.at[pl.ds(base, CH)], st[b])

        def store_drain(b):
            pltpu.make_async_copy(bz.at[b], z1_ref.at[pl.ds(0, CH)], st[b]).wait()
            pltpu.make_async_copy(bx.at[b], dx_ref.at[pl.ds(0, CH)], st[b]).wait()

        a_start(0, 0)

        def outer(g, carry):
            for b0 in (0, 1, 2):
                t = 3 * g + b0
                bt = b0
                bp = (b0 + 2) % 3
                bq = (b0 + 1) % 3

                @pl.when(valid(t - 2))
                def _():
                    store_drain(bq)

                @pl.when(valid(t + 1))
                def _():
                    a_start(t + 1, bq)

                @pl.when(valid(t))
                def _():
                    a_drain(bt)
                    b_start(bt)

                @pl.when(valid(t - 1))
                def _():
                    b_drain(bp)
                    store_start(t - 1, bp)

            return carry

        lax.fori_loop(0, (trips + 3) // 3 + 1, outer, 0)

    return k(pr, pc, x16, xneg, row, col)


def _sc_scatter(ef, tr, row, z128, z16):
    eh = row.shape[0]
    nch = eh // CH
    trips = -(-nch // NW)

    @functools.partial(
        pl.kernel,
        out_type=(
            jax.ShapeDtypeStruct((NC, N, HID), jnp.float32),
            jax.ShapeDtypeStruct((NC, N, XW), jnp.float32),
        ),
        mesh=plsc.VectorSubcoreMesh(core_axis_name="c", subcore_axis_name="s"),
        scratch_types=(
            pltpu.VMEM_SHARED((N, HID), jnp.float32),
            pltpu.VMEM_SHARED((N, XW), jnp.float32),
            pltpu.VMEM((2, CH), jnp.int32),
            pltpu.VMEM((2, CH, HID), jnp.float32),
            pltpu.VMEM((2, CH, XW), jnp.float32),
            pltpu.SemaphoreType.DMA,
            pltpu.SemaphoreType.DMA,
            pltpu.SemaphoreType.DMA,
            pltpu.SemaphoreType.DMA,
        ),
        compiler_params=pltpu.CompilerParams(use_tc_tiling_on_sc=False),
    )
    def k(ef_ref, tr_ref, row_ref, z128_ref, z16_ref, agg_ref, tagg_ref,
          acc, tacc, ir2, bef, btr, ls0, ls1, as0, as1):
        c = lax.axis_index("c")
        s = lax.axis_index("s")
        w = s * NC + c
        r0 = s * RPS
        pltpu.sync_copy(z128_ref, acc.at[pl.ds(r0, RPS)])
        pltpu.sync_copy(z16_ref, tacc.at[pl.ds(r0, RPS)])
        plsc.subcore_barrier()
        ls = (ls0, ls1)
        am = (as0, as1)

        def valid(t):
            return (t >= 0) & (w + t * NW < nch)

        def load_start(t, b):
            base = (w + t * NW) * CH
            pltpu.async_copy(row_ref.at[pl.ds(base, CH)], ir2.at[b], ls[b])
            pltpu.async_copy(ef_ref.at[pl.ds(base, CH)], bef.at[b], ls[b])
            pltpu.async_copy(tr_ref.at[pl.ds(base, CH)], btr.at[b], ls[b])

        def load_drain(b):
            pltpu.make_async_copy(row_ref.at[pl.ds(0, CH)], ir2.at[b], ls[b]).wait()
            pltpu.make_async_copy(ef_ref.at[pl.ds(0, CH)], bef.at[b], ls[b]).wait()
            pltpu.make_async_copy(tr_ref.at[pl.ds(0, CH)], btr.at[b], ls[b]).wait()

        def add_start(b):
            pltpu.async_copy(bef.at[b], acc.at[ir2.at[b]], am[b], add=True)
            pltpu.async_copy(btr.at[b], tacc.at[ir2.at[b]], am[b], add=True)

        def add_drain(b):
            pltpu.make_async_copy(bef.at[b], acc.at[pl.ds(0, CH)], am[b]).wait()
            pltpu.make_async_copy(btr.at[b], tacc.at[pl.ds(0, CH)], am[b]).wait()

        load_start(0, 0)

        def outer(g, carry):
            for b in (0, 1):
                t = 2 * g + b
                o = 1 - b

                @pl.when(valid(t - 1))
                def _():
                    add_drain(o)

                @pl.when(valid(t + 1))
                def _():
                    load_start(t + 1, o)

                @pl.when(valid(t))
                def _():
                    load_drain(b)
                    add_start(b)

            return carry

        lax.fori_loop(0, (trips + 1) // 2, outer, 0)
        plsc.subcore_barrier()
        pltpu.sync_copy(acc.at[pl.ds(r0, RPS)], agg_ref.at[c, pl.ds(r0, RPS)])
        pltpu.sync_copy(tacc.at[pl.ds(r0, RPS)], tagg_ref.at[c, pl.ds(r0, RPS)])

    return k(ef, tr, row, z128, z16)


def _full(shape):
    return pl.BlockSpec(shape, lambda i: tuple(0 for _ in shape))


def _tc_embed(h, x16, emb_w, emb_b, w1a, w1b, b1):
    def body(h_ref, x_ref, ew_r, eb_r, w1a_r, w1b_r, b1_r,
             hh_ref, pr_ref, pc_ref, xn_ref):
        hh = jnp.dot(h_ref[...], ew_r[...], preferred_element_type=jnp.float32) + eb_r[...]
        hh_ref[...] = hh
        pr_ref[...] = jnp.dot(hh, w1a_r[...], preferred_element_type=jnp.float32) + b1_r[...]
        pc_ref[...] = jnp.dot(hh, w1b_r[...], preferred_element_type=jnp.float32)
        xn_ref[...] = -x_ref[...]

    return pl.pallas_call(
        body,
        grid=(N // BN,),
        in_specs=[
            pl.BlockSpec((BN, D_IN), lambda i: (i, 0)),
            pl.BlockSpec((BN, XW), lambda i: (i, 0)),
            _full((D_IN, HID)),
            _full((1, HID)),
            _full((HID, HID)),
            _full((HID, HID)),
            _full((1, HID)),
        ],
        out_specs=[
            pl.BlockSpec((BN, HID), lambda i: (i, 0)),
            pl.BlockSpec((BN, HID), lambda i: (i, 0)),
            pl.BlockSpec((BN, HID), lambda i: (i, 0)),
            pl.BlockSpec((BN, XW), lambda i: (i, 0)),
        ],
        out_shape=[
            jax.ShapeDtypeStruct((N, HID), jnp.float32),
            jax.ShapeDtypeStruct((N, HID), jnp.float32),
            jax.ShapeDtypeStruct((N, HID), jnp.float32),
            jax.ShapeDtypeStruct((N, XW), jnp.float32),
        ],
    )(h, x16, emb_w, emb_b.reshape(1, HID), w1a, w1b, b1.reshape(1, HID))


def _tc_edge(z1, dx, ea, wts):
    (w1r, w1e, w2, b2, aw, ab, cw1, cb1, cw2) = wts
    eh = z1.shape[0]

    def body(z1_ref, dx_ref, ea_ref,
             w1r_r, w1e_r, w2_r, b2_r, aw_r, ab_r, cw1_r, cb1_r, cw2_r,
             ef_ref, tr_ref):
        diff = dx_ref[...]
        radial = jnp.dot(diff * diff, jnp.ones((XW, 1), jnp.float32),
                         preferred_element_type=jnp.float32)
        z = (
            z1_ref[...]
            + jnp.dot(ea_ref[...], w1e_r[...], preferred_element_type=jnp.float32)
            + radial * w1r_r[...]
        )
        z = _silu(z)
        z = _silu(jnp.dot(z, w2_r[...], preferred_element_type=jnp.float32) + b2_r[...])
        att = _sigmoid(jnp.dot(z, aw_r[...], preferred_element_type=jnp.float32)
                       + ab_r[...])
        efv = z * att
        t = _silu(jnp.dot(efv, cw1_r[...], preferred_element_type=jnp.float32) + cb1_r[...])
        ts = jnp.tanh(jnp.dot(t, cw2_r[...], preferred_element_type=jnp.float32))
        lane = lax.broadcasted_iota(jnp.int32, (BE, XW), 1)
        ef_ref[...] = efv
        tr_ref[...] = jnp.where(lane == 3, 1.0, diff * ts)

    return pl.pallas_call(
        body,
        grid=(eh // BE,),
        in_specs=[
            pl.BlockSpec((BE, HID), lambda i: (i, 0)),
            pl.BlockSpec((BE, XW), lambda i: (i, 0)),
            pl.BlockSpec((BE, D_EDGE), lambda i: (i, 0)),
            _full((1, HID)),
            _full((D_EDGE, HID)),
            _full((HID, HID)),
            _full((1, HID)),
            _full((HID, 1)),
            _full((1, 1)),
            _full((HID, HID)),
            _full((1, HID)),
            _full((HID, 1)),
        ],
        out_specs=[
            pl.BlockSpec((BE, HID), lambda i: (i, 0)),
            pl.BlockSpec((BE, XW), lambda i: (i, 0)),
        ],
        out_shape=[
            jax.ShapeDtypeStruct((eh, HID), jnp.float32),
            jax.ShapeDtypeStruct((eh, XW), jnp.float32),
        ],
    )(z1, dx, ea, w1r, w1e, w2, b2, aw, ab, cw1, cb1, cw2)


def _tc_node(hh, x16, aggs, taggs, wts, nxt):
    (nw1a, nw1b, nb1, nw2, nb2) = wts
    last = len(nxt) == 2
    nh_parts = len(aggs)
    npart = 2 * nh_parts

    def _hh2(hh_ref, a_refs, nw1a_r, nw1b_r, nb1_r, nw2_r, nb2_r):
        aggv = a_refs[0][0]
        for a in a_refs[1:]:
            aggv = aggv + a[0]
        nh = _silu(
            jnp.dot(hh_ref[...], nw1a_r[...], preferred_element_type=jnp.float32)
            + jnp.dot(aggv, nw1b_r[...], preferred_element_type=jnp.float32)
            + nb1_r[...]
        )
        return (
            hh_ref[...]
            + jnp.dot(nh, nw2_r[...], preferred_element_type=jnp.float32)
            + nb2_r[...]
        )

    def body_last(*refs):
        hh_ref = refs[0]
        a_refs = refs[1:1 + npart]
        (x_ref, nw1a_r, nw1b_r, nb1_r, nw2_r, nb2_r, wo_r, bo_r, z_ref) = \
            refs[1 + 2 * npart:]
        hh2 = _hh2(hh_ref, a_refs, nw1a_r, nw1b_r, nb1_r, nw2_r, nb2_r)
        z_ref[...] = (
            jnp.dot(hh2, wo_r[...], preferred_element_type=jnp.float32) + bo_r[...]
        )

    def body_mid(*refs):
        hh_ref = refs[0]
        a_refs = refs[1:1 + npart]
        t_refs = refs[1 + npart:1 + 2 * npart]
        (x_ref, nw1a_r, nw1b_r, nb1_r, nw2_r, nb2_r, pa_r, pcw_r, pb_r,
         hho_ref, xo_ref, xno_ref, pr_ref, pc_ref) = refs[1 + 2 * npart:]
        hh2 = _hh2(hh_ref, a_refs, nw1a_r, nw1b_r, nb1_r, nw2_r, nb2_r)
        ts = t_refs[0][0]
        for t in t_refs[1:]:
            ts = ts + t[0]
        cnt = ts[:, 3:4]
        inv = 1.0 / jnp.maximum(cnt, 1.0)
        lane = lax.broadcasted_iota(jnp.int32, (BN, XW), 1)
        xo = x_ref[...] + jnp.where(lane < 3, ts * inv, 0.0)
        hho_ref[...] = hh2
        xo_ref[...] = xo
        xno_ref[...] = -xo
        pr_ref[...] = (
            jnp.dot(hh2, pa_r[...], preferred_element_type=jnp.float32) + pb_r[...]
        )
        pc_ref[...] = jnp.dot(hh2, pcw_r[...], preferred_element_type=jnp.float32)

    wrapped_body = body_last if last else body_mid

    def _part_spec(core, width):
        return pl.BlockSpec((1, BN, width), lambda i, c=core: (c, i, 0))

    agg_specs = []
    agg_args = []
    for a in aggs:
        for core in range(NC):
            agg_specs.append(_part_spec(core, HID))
            agg_args.append(a)
    tagg_specs = []
    tagg_args = []
    for t in taggs:
        for core in range(NC):
            tagg_specs.append(_part_spec(core, XW))
            tagg_args.append(t)

    base_in = (
        [pl.BlockSpec((BN, HID), lambda i: (i, 0))]
        + agg_specs
        + tagg_specs
        + [
            pl.BlockSpec((BN, XW), lambda i: (i, 0)),
            _full((HID, HID)),
            _full((HID, HID)),
            _full((1, HID)),
            _full((HID, HID)),
            _full((1, HID)),
        ]
    )
    base_args = [hh] + agg_args + tagg_args + [
        x16, nw1a, nw1b, nb1.reshape(1, HID), nw2, nb2.reshape(1, HID)]
    if last:
        w_out, b_out = nxt
        in_specs = base_in + [_full((HID, HID)), _full((1, HID))]
        out_specs = [pl.BlockSpec((BN, HID), lambda i: (i, 0))]
        out_shape = [jax.ShapeDtypeStruct((N, HID), jnp.float32)]
        args = tuple(base_args + [w_out, b_out.reshape(1, HID)])
    else:
        w1a_n, w1b_n, b1_n = nxt
        in_specs = base_in + [_full((HID, HID)), _full((HID, HID)), _full((1, HID))]
        out_specs = [
            pl.BlockSpec((BN, HID), lambda i: (i, 0)),
            pl.BlockSpec((BN, XW), lambda i: (i, 0)),
            pl.BlockSpec((BN, XW), lambda i: (i, 0)),
            pl.BlockSpec((BN, HID), lambda i: (i, 0)),
            pl.BlockSpec((BN, HID), lambda i: (i, 0)),
        ]
        out_shape = [
            jax.ShapeDtypeStruct((N, HID), jnp.float32),
            jax.ShapeDtypeStruct((N, XW), jnp.float32),
            jax.ShapeDtypeStruct((N, XW), jnp.float32),
            jax.ShapeDtypeStruct((N, HID), jnp.float32),
            jax.ShapeDtypeStruct((N, HID), jnp.float32),
        ]
        args = tuple(base_args + [w1a_n, w1b_n, b1_n.reshape(1, HID)])

    return pl.pallas_call(
        wrapped_body,
        grid=(N // BN,),
        in_specs=in_specs,
        out_specs=out_specs,
        out_shape=out_shape,
    )(*args)


def _tc_pool(z, batch3):
    def body(z_ref, bt_ref, o_ref, acc, cacc):
        i = pl.program_id(0)

        @pl.when(i == 0)
        def _():
            acc[...] = jnp.zeros_like(acc)
            cacc[...] = jnp.zeros_like(cacc)

        gid = lax.broadcasted_iota(jnp.int32, (NG, BP), 0)
        oh = (gid == bt_ref[0]).astype(jnp.float32)
        acc[...] += jnp.dot(oh, z_ref[...], preferred_element_type=jnp.float32)
        cacc[...] += jnp.sum(oh, axis=1, keepdims=True)

        @pl.when(i == GP - 1)
        def _():
            o_ref[...] = acc[...] / jnp.maximum(cacc[...], 1.0)

    return pl.pallas_call(
        body,
        grid=(GP,),
        in_specs=[
            pl.BlockSpec((BP, HID), lambda i: (i, 0)),
            pl.BlockSpec((1, 1, BP), lambda i: (i, 0, 0)),
        ],
        out_specs=pl.BlockSpec((NG, HID), lambda i: (0, 0)),
        out_shape=jax.ShapeDtypeStruct((NG, HID), jnp.float32),
        scratch_shapes=[
            pltpu.VMEM((NG, HID), jnp.float32),
            pltpu.VMEM((NG, 1), jnp.float32),
        ],
    )(z, batch3)


def _edge_split(l):
    ew1 = l["edge_w1"]
    return ew1[:HID], ew1[HID:2 * HID], ew1[2 * HID:2 * HID + 1], ew1[2 * HID + 1:]


def kernel(h, x, edge_attr, params, edges, batch):
    row = edges[0]
    col = edges[1]
    eh = E // 2
    halves = (
        (row[:eh], col[:eh], edge_attr[:eh]),
        (row[eh:], col[eh:], edge_attr[eh:]),
    )
    x16 = jnp.zeros((N, XW), jnp.float32).at[:, :3].set(x)
    z128 = jnp.zeros((RPS, HID), jnp.float32)
    z16 = jnp.zeros((RPS, XW), jnp.float32)
    batch3 = batch.reshape(GP, 1, BP)
    layers = params["layers"]

    w1a0, w1b0, _, _ = _edge_split(layers[0])
    hh, pr, pc, xneg = _tc_embed(
        h, x16, params["emb_in_w"], params["emb_in_b"], w1a0, w1b0,
        layers[0]["edge_b1"])

    for li, l in enumerate(layers):
        _, _, w1r, w1e = _edge_split(l)
        edge_wts = (
            w1r,
            w1e,
            l["edge_w2"],
            l["edge_b2"].reshape(1, HID),
            l["att_w"],
            l["att_b"].reshape(1, 1),
            l["coord_w1"],
            l["coord_b1"].reshape(1, HID),
            l["coord_w2"],
        )
        node_wts = (
            l["node_w1"][:HID],
            l["node_w1"][HID:],
            l["node_b1"],
            l["node_w2"],
            l["node_b2"],
        )
        if li + 1 < len(layers):
            nl = layers[li + 1]
            w1a_n, w1b_n, _, _ = _edge_split(nl)
            nxt = (w1a_n, w1b_n, nl["edge_b1"])
        else:
            nxt = (params["emb_out_w"], params["emb_out_b"])

        aggs = []
        taggs = []
        efts = []
        for hrow, hcol, hea in halves:
            z1, dx = _sc_gather(pr, pc, x16, xneg, hrow, hcol)
            efts.append(_tc_edge(z1, dx, hea, edge_wts))
        for (hrow, _, _), (ef, tr) in zip(halves, efts):
            agg, tagg = _sc_scatter(ef, tr, hrow, z128, z16)
            aggs.append(agg)
            taggs.append(tagg)
        outs = _tc_node(hh, x16, tuple(aggs), tuple(taggs), node_wts, nxt)
        if li + 1 < len(layers):
            hh, x16, xneg, pr, pc = outs
        else:
            (zfin,) = outs

    return _tc_pool(zfin, batch3)
